# baseline (speedup 1.0000x reference)
import functools

import jax
import jax.numpy as jnp
from jax import lax
from jax.experimental import pallas as pl
from jax.experimental.pallas import tpu as pltpu
from jax.experimental.pallas import tpu_sc as plsc

_N = 10000
_E = 320000
_R = 8
_B = 4

_NC = 2
_NS = 16
_NW = _NC * _NS
_EPW = _E // _NW
_K = 80
_CPT = _EPW // _K
_RPT = 624
_RREM = _N - _NS * _RPT

_BM = 1000


def _emit_transform(h, w_ref, c_ref, lw_ref, t_ref, loop_ref):
    bases = [
        jnp.dot(h, w_ref[b], preferred_element_type=jnp.float32)
        for b in range(_B)
    ]
    for r in range(_R):
        acc = c_ref[r, 0] * bases[0]
        for b in range(1, _B):
            acc = acc + c_ref[r, b] * bases[b]
        t_ref[r] = acc.astype(t_ref.dtype)
    loop_ref[...] = jnp.dot(h, lw_ref[...], preferred_element_type=jnp.float32)


def _xform_first_body(x_ref, w_ref, c_ref, lw_ref, t_ref, loop_ref):
    _emit_transform(x_ref[...], w_ref, c_ref, lw_ref, t_ref, loop_ref)


def _xform_mid_body(acc_ref, lp_ref, b_ref, w_ref, c_ref, lw_ref, t_ref,
                    loop_ref):
    h = jnp.maximum(acc_ref[0] + acc_ref[1] + lp_ref[...] + b_ref[...], 0.0)
    _emit_transform(h, w_ref, c_ref, lw_ref, t_ref, loop_ref)


def _final_body(acc_ref, lp_ref, b_ref, o_ref):
    o_ref[...] = acc_ref[0] + acc_ref[1] + lp_ref[...] + b_ref[...]


def _t_out(do, t_dtype):
    return (
        [pl.BlockSpec((_R, _BM, do), lambda i: (0, i, 0)),
         pl.BlockSpec((_BM, do), lambda i: (i, 0))],
        [jax.ShapeDtypeStruct((_R, _N, do), t_dtype),
         jax.ShapeDtypeStruct((_N, do), jnp.float32)],
    )


def _make_xform_first(di, do, t_dtype=jnp.float32):
    out_specs, out_shape = _t_out(do, t_dtype)
    return pl.pallas_call(
        _xform_first_body,
        grid=(_N // _BM,),
        in_specs=[
            pl.BlockSpec((_BM, di), lambda i: (i, 0)),
            pl.BlockSpec((_B, di, do), lambda i: (0, 0, 0)),
            pl.BlockSpec(memory_space=pltpu.SMEM),
            pl.BlockSpec((di, do), lambda i: (0, 0)),
        ],
        out_specs=out_specs,
        out_shape=out_shape,
    )


def _make_xform_mid(di, do, t_dtype=jnp.float32):
    out_specs, out_shape = _t_out(do, t_dtype)
    return pl.pallas_call(
        _xform_mid_body,
        grid=(_N // _BM,),
        in_specs=[
            pl.BlockSpec((_NC, _BM, di), lambda i: (0, i, 0)),
            pl.BlockSpec((_BM, di), lambda i: (i, 0)),
            pl.BlockSpec((1, di), lambda i: (0, 0)),
            pl.BlockSpec((_B, di, do), lambda i: (0, 0, 0)),
            pl.BlockSpec(memory_space=pltpu.SMEM),
            pl.BlockSpec((di, do), lambda i: (0, 0)),
        ],
        out_specs=out_specs,
        out_shape=out_shape,
    )


def _make_final(do):
    return pl.pallas_call(
        _final_body,
        grid=(_N // _BM,),
        in_specs=[
            pl.BlockSpec((_NC, _BM, do), lambda i: (0, i, 0)),
            pl.BlockSpec((_BM, do), lambda i: (i, 0)),
            pl.BlockSpec((1, do), lambda i: (0, 0)),
        ],
        out_specs=pl.BlockSpec((_BM, do), lambda i: (i, 0)),
        out_shape=jax.ShapeDtypeStruct((_N, do), jnp.float32),
    )


def _make_sc_agg(do):
    nsl = do // 16
    mesh = plsc.VectorSubcoreMesh(core_axis_name="c", subcore_axis_name="s",
                                  num_cores=_NC, num_subcores=_NS)

    @functools.partial(
        pl.kernel,
        out_type=jax.ShapeDtypeStruct((_NC, _N, do), jnp.float32),
        mesh=mesh,
        scratch_types=[
            pltpu.VMEM((4, 2, _K), jnp.int32),
            pltpu.VMEM((4, _K), jnp.int32),
            pltpu.VMEM((_K, do), jnp.float32),
            pltpu.VMEM((_K, do), jnp.float32),
            pltpu.VMEM((_K, do), jnp.float32),
            pltpu.VMEM((_K, do), jnp.float32),
            pltpu.VMEM_SHARED((_N, do), jnp.float32),
            pltpu.SemaphoreType.DMA,
            pltpu.SemaphoreType.DMA,
            pltpu.SemaphoreType.DMA,
            pltpu.SemaphoreType.DMA,
            pltpu.SemaphoreType.DMA,
            pltpu.SemaphoreType.DMA,
            pltpu.SemaphoreType.DMA,
            pltpu.SemaphoreType.DMA,
            pltpu.SemaphoreType.DMA,
            pltpu.SemaphoreType.DMA,
            pltpu.SemaphoreType.DMA,
            pltpu.SemaphoreType.DMA,
        ],
        compiler_params=pltpu.CompilerParams(use_tc_tiling_on_sc=False,
                                             needs_layout_passes=False),
    )
    def agg(table, metas, zeros, out, meta_v, dst_v,
            rows0, rows1, rows2, rows3, acc_sh,
            gsem0, gsem1, gsem2, gsem3, ssem0, ssem1, ssem2, ssem3,
            msem0, msem1, msem2, msem3):
        c = lax.axis_index("c")
        s = lax.axis_index("s")
        wid = s * _NC + c
        rowbase = wid * _CPT

        rows = (rows0, rows1, rows2, rows3)
        gsem = (gsem0, gsem1, gsem2, gsem3)
        ssem = (ssem0, ssem1, ssem2, ssem3)
        msem = (msem0, msem1, msem2, msem3)

        pltpu.sync_copy(zeros.at[pl.ds(s * _RPT, _RPT)],
                        acc_sh.at[pl.ds(s * _RPT, _RPT)])

        @pl.when(s == _NS - 1)
        def _zero_rem():
            pltpu.sync_copy(zeros.at[pl.ds(_NS * _RPT, _RREM)],
                            acc_sh.at[pl.ds(_NS * _RPT, _RREM)])

        plsc.subcore_barrier()

        def issue_meta(ci, slot):
            pltpu.async_copy(metas.at[rowbase + ci], meta_v.at[slot],
                             msem[slot])

        def wait_meta(slot):
            pltpu.make_async_copy(metas.at[0], meta_v.at[slot],
                                  msem[slot]).wait()

        def issue_gather(slot):
            pltpu.async_copy(table.at[meta_v.at[slot, 0]], rows[slot],
                             gsem[slot])

        def wait_gather(b):
            pltpu.make_async_copy(table.at[meta_v.at[0, 0]], rows[b],
                                  gsem[b]).wait()

        def issue_scatter(b):
            pltpu.async_copy(rows[b], acc_sh.at[dst_v.at[b]], ssem[b],
                             add=True)

        def wait_scatter(b):
            pltpu.make_async_copy(rows[b], acc_sh.at[dst_v.at[0]],
                                  ssem[b]).wait()

        def scale_and_unpack(b):
            buf = rows[b]

            def rowgroup(g, carry):
                pk = meta_v[b, 1, pl.ds(g * 16, 16)]
                dst_v[b, pl.ds(g * 16, 16)] = pk & jnp.int32(0xFFFF)
                nv = plsc.bitcast(pk & jnp.int32(-65536), jnp.float32)
                for t in range(16):
                    i_row = g * 16 + t
                    sn = nv[t]
                    for j in range(nsl):
                        sl = pl.ds(j * 16, 16)
                        buf[i_row, sl] = buf[i_row, sl] * sn
                return carry

            lax.fori_loop(0, _K // 16, rowgroup, 0)

        for slot in range(4):
            issue_meta(slot, slot)
        wait_meta(0)
        issue_gather(0)
        wait_meta(1)
        issue_gather(1)

        def quad(j, carry):
            c0 = j * 4
            for b in range(4):
                ci = c0 + b
                wait_gather(b)
                scale_and_unpack(b)
                issue_scatter(b)

                @pl.when(ci + 4 <= _CPT - 1)
                def _():
                    issue_meta(ci + 4, b)

                if b >= 2:
                    wait_scatter(b - 2)
                else:
                    @pl.when(j >= 1)
                    def _():
                        wait_scatter((b + 2) % 4)
                if b == 3:
                    @pl.when(j <= (_CPT - 1) // 4 - 2)
                    def _():
                        wait_meta((b + 2) % 4)
                        issue_gather((b + 2) % 4)
                else:
                    wait_meta((b + 2) % 4)
                    issue_gather((b + 2) % 4)
            return carry

        lax.fori_loop(0, (_CPT - 1) // 4, quad, 0)

        wait_gather(0)
        scale_and_unpack(0)
        pltpu.sync_copy(rows[0], acc_sh.at[dst_v.at[0]], add=True)
        wait_scatter(2)
        wait_scatter(3)

        plsc.subcore_barrier()
        pltpu.sync_copy(acc_sh.at[pl.ds(s * _RPT, _RPT)],
                        out.at[c, pl.ds(s * _RPT, _RPT)])

        @pl.when(s == _NS - 1)
        def _out_rem():
            pltpu.sync_copy(acc_sh.at[pl.ds(_NS * _RPT, _RREM)],
                            out.at[c, pl.ds(_NS * _RPT, _RREM)])

    return agg


def _make_sc_agg_packed():
    do = 128
    dow = do // 2
    mesh = plsc.VectorSubcoreMesh(core_axis_name="c", subcore_axis_name="s",
                                  num_cores=_NC, num_subcores=_NS)

    @functools.partial(
        pl.kernel,
        out_type=jax.ShapeDtypeStruct((_NC, _N, do), jnp.float32),
        mesh=mesh,
        scratch_types=[
            pltpu.VMEM((4, 2, _K), jnp.int32),
            pltpu.VMEM((4, _K), jnp.int32),
            pltpu.VMEM((_K, dow), jnp.int32),
            pltpu.VMEM((_K, dow), jnp.int32),
            pltpu.VMEM((_K, dow), jnp.int32),
            pltpu.VMEM((_K, dow), jnp.int32),
            pltpu.VMEM((_K, do), jnp.float32),
            pltpu.VMEM((_K, do), jnp.float32),
            pltpu.VMEM_SHARED((_N, do), jnp.float32),
            pltpu.SemaphoreType.DMA,
            pltpu.SemaphoreType.DMA,
            pltpu.SemaphoreType.DMA,
            pltpu.SemaphoreType.DMA,
            pltpu.SemaphoreType.DMA,
            pltpu.SemaphoreType.DMA,
            pltpu.SemaphoreType.DMA,
            pltpu.SemaphoreType.DMA,
            pltpu.SemaphoreType.DMA,
            pltpu.SemaphoreType.DMA,
        ],
        compiler_params=pltpu.CompilerParams(use_tc_tiling_on_sc=False,
                                             needs_layout_passes=False),
    )
    def agg(table, metas, zeros, out, meta_v, dst_v,
            rows0, rows1, rows2, rows3, sbuf0, sbuf1, acc_sh,
            gsem0, gsem1, gsem2, gsem3, msem0, msem1, msem2, msem3,
            ssem0, ssem1):
        c = lax.axis_index("c")
        s = lax.axis_index("s")
        wid = s * _NC + c
        rowbase = wid * _CPT

        rows = (rows0, rows1, rows2, rows3)
        sbuf = (sbuf0, sbuf1)
        gsem = (gsem0, gsem1, gsem2, gsem3)
        msem = (msem0, msem1, msem2, msem3)
        ssem = (ssem0, ssem1)

        pltpu.sync_copy(zeros.at[pl.ds(s * _RPT, _RPT)],
                        acc_sh.at[pl.ds(s * _RPT, _RPT)])

        @pl.when(s == _NS - 1)
        def _zero_rem():
            pltpu.sync_copy(zeros.at[pl.ds(_NS * _RPT, _RREM)],
                            acc_sh.at[pl.ds(_NS * _RPT, _RREM)])

        plsc.subcore_barrier()

        def issue_meta(ci, slot):
            pltpu.async_copy(metas.at[rowbase + ci], meta_v.at[slot],
                             msem[slot])

        def wait_meta(slot):
            pltpu.make_async_copy(metas.at[0], meta_v.at[slot],
                                  msem[slot]).wait()

        def issue_gather(slot):
            pltpu.async_copy(table.at[meta_v.at[slot, 0]], rows[slot],
                             gsem[slot])

        def wait_gather(b):
            pltpu.make_async_copy(table.at[meta_v.at[0, 0]], rows[b],
                                  gsem[b]).wait()

        def issue_scatter(b, sb):
            pltpu.async_copy(sbuf[sb], acc_sh.at[dst_v.at[b]], ssem[sb],
                             add=True)

        def wait_scatter(sb):
            pltpu.make_async_copy(sbuf[sb], acc_sh.at[dst_v.at[0]],
                                  ssem[sb]).wait()

        def scale_unpack(b, sb):
            src = rows[b]
            dstb = sbuf[sb]

            def rowgroup(g, carry):
                pkm = meta_v[b, 1, pl.ds(g * 16, 16)]
                dst_v[b, pl.ds(g * 16, 16)] = pkm & jnp.int32(0xFFFF)
                nv = plsc.bitcast(pkm & jnp.int32(-65536), jnp.float32)
                for t in range(16):
                    i_row = g * 16 + t
                    sn = nv[t]
                    for j in range(do // 32):
                        pk = src[i_row, pl.ds(j * 16, 16)]
                        lo = plsc.bitcast(pk << 16, jnp.float32)
                        hi = plsc.bitcast(pk & jnp.int32(-65536), jnp.float32)
                        dstb[i_row, pl.ds(j * 32, 16)] = lo * sn
                        dstb[i_row, pl.ds(j * 32 + 16, 16)] = hi * sn
                return carry

            lax.fori_loop(0, _K // 16, rowgroup, 0)

        for slot in range(4):
            issue_meta(slot, slot)
        wait_meta(0)
        issue_gather(0)
        wait_meta(1)
        issue_gather(1)

        def quad(j, carry):
            c0 = j * 4
            for b in range(4):
                ci = c0 + b
                sb = b % 2
                if b == 3:
                    @pl.when(j <= (_CPT - 1) // 4 - 2)
                    def _():
                        wait_meta((b + 2) % 4)
                        issue_gather((b + 2) % 4)
                else:
                    wait_meta((b + 2) % 4)
                    issue_gather((b + 2) % 4)
                wait_gather(b)
                if b >= 2:
                    wait_scatter(sb)
                else:
                    @pl.when(j >= 1)
                    def _():
                        wait_scatter(sb)
                scale_unpack(b, sb)
                issue_scatter(b, sb)

                @pl.when(ci + 4 <= _CPT - 1)
                def _():
                    issue_meta(ci + 4, b)
            return carry

        lax.fori_loop(0, (_CPT - 1) // 4, quad, 0)

        wait_gather(0)
        wait_scatter(0)
        scale_unpack(0, 0)
        pltpu.sync_copy(sbuf[0], acc_sh.at[dst_v.at[0]], add=True)
        wait_scatter(1)

        plsc.subcore_barrier()
        pltpu.sync_copy(acc_sh.at[pl.ds(s * _RPT, _RPT)],
                        out.at[c, pl.ds(s * _RPT, _RPT)])

        @pl.when(s == _NS - 1)
        def _out_rem():
            pltpu.sync_copy(acc_sh.at[pl.ds(_NS * _RPT, _RREM)],
                            out.at[c, pl.ds(_NS * _RPT, _RREM)])

    return agg


_COLMAP = []
for _g in range(4):
    for _i in range(16):
        _COLMAP.extend([32 * _g + _i, 32 * _g + 16 + _i])
_COLMAP = tuple(_COLMAP)

_xform0 = _make_xform_first(128, 128, jnp.bfloat16)
_xform1 = _make_xform_mid(128, 128, jnp.bfloat16)
_xform2 = _make_xform_mid(128, 16)
_final = _make_final(16)
_make_sc_agg = functools.lru_cache(maxsize=None)(_make_sc_agg)


def kernel(x, edge_index, edge_type, edge_norm,
           W0, C0, LW0, b0, W1, C1, LW1, b1, W2, C2, LW2, b2):
    src = edge_index[0].astype(jnp.int32)
    dst = edge_index[1].astype(jnp.int32)
    et = edge_type.astype(jnp.int32)
    flat_idx = (et * _N + src).reshape(_E // _K, _K)
    nbits = lax.bitcast_convert_type(
        edge_norm.reshape(-1).astype(jnp.bfloat16), jnp.uint16)
    packed = (nbits.astype(jnp.uint32) << 16) | dst.astype(jnp.uint32)
    packed = lax.bitcast_convert_type(packed, jnp.int32).reshape(
        _E // _K, _K)
    metas = jnp.stack([flat_idx, packed], axis=1)
    z128 = jnp.zeros((_N, 128), jnp.float32)
    z16 = jnp.zeros((_N, 16), jnp.float32)

    sc_agg_128 = _make_sc_agg_packed()
    sc_agg_16 = _make_sc_agg(16)

    cm = jnp.asarray(_COLMAP, jnp.int32)
    W0p = W0[:, :, cm]
    W1p = W1[:, :, cm]

    def _pack_table(t):
        return lax.bitcast_convert_type(
            t.reshape(_R * _N, 64, 2), jnp.int32)

    t0, lp0 = _xform0(x, W0p, C0, LW0)
    acc0 = sc_agg_128(_pack_table(t0), metas, z128)

    t1, lp1 = _xform1(acc0, lp0, b0.reshape(1, -1), W1p, C1, LW1)
    acc1 = sc_agg_128(_pack_table(t1), metas, z128)

    t2, lp2 = _xform2(acc1, lp1, b1.reshape(1, -1), W2, C2, LW2)
    acc2 = sc_agg_16(t2.reshape(_R * _N, 16), metas, z16)

    return _final(acc2, lp2, b2.reshape(1, -1))


# speedup vs baseline: 2.5384x; 2.5384x over previous
import functools

import jax
import jax.numpy as jnp
from jax import lax
from jax.experimental import pallas as pl
from jax.experimental.pallas import tpu as pltpu
from jax.experimental.pallas import tpu_sc as plsc

_N = 10000
_E = 320000
_R = 8
_B = 4

_NC = 2
_NS = 16
_NW = _NC * _NS
_EPW = _E // _NW
_K = 80
_CPT = _EPW // _K
_RPT = 624
_RREM = _N - _NS * _RPT

_BM = 1000


def _emit_transform(h, w_ref, c_ref, lw_ref, t_ref, loop_ref):
    bases = [
        jnp.dot(h, w_ref[b], preferred_element_type=jnp.float32)
        for b in range(_B)
    ]
    for r in range(_R):
        acc = c_ref[r, 0] * bases[0]
        for b in range(1, _B):
            acc = acc + c_ref[r, b] * bases[b]
        t_ref[r] = acc.astype(t_ref.dtype)
    loop_ref[...] = jnp.dot(h, lw_ref[...], preferred_element_type=jnp.float32)


def _xform_first_body(x_ref, w_ref, c_ref, lw_ref, t_ref, loop_ref):
    _emit_transform(x_ref[...], w_ref, c_ref, lw_ref, t_ref, loop_ref)


def _xform_mid_body(acc_ref, lp_ref, b_ref, w_ref, c_ref, lw_ref, t_ref,
                    loop_ref):
    h = jnp.maximum(acc_ref[0] + acc_ref[1] + lp_ref[...] + b_ref[...], 0.0)
    _emit_transform(h, w_ref, c_ref, lw_ref, t_ref, loop_ref)


def _final_body(acc_ref, lp_ref, b_ref, o_ref):
    o_ref[...] = acc_ref[0] + acc_ref[1] + lp_ref[...] + b_ref[...]


def _t_out(do, t_dtype):
    return (
        [pl.BlockSpec((_R, _BM, do), lambda i: (0, i, 0)),
         pl.BlockSpec((_BM, do), lambda i: (i, 0))],
        [jax.ShapeDtypeStruct((_R, _N, do), t_dtype),
         jax.ShapeDtypeStruct((_N, do), jnp.float32)],
    )


def _make_xform_first(di, do, t_dtype=jnp.float32):
    out_specs, out_shape = _t_out(do, t_dtype)
    return pl.pallas_call(
        _xform_first_body,
        grid=(_N // _BM,),
        in_specs=[
            pl.BlockSpec((_BM, di), lambda i: (i, 0)),
            pl.BlockSpec((_B, di, do), lambda i: (0, 0, 0)),
            pl.BlockSpec(memory_space=pltpu.SMEM),
            pl.BlockSpec((di, do), lambda i: (0, 0)),
        ],
        out_specs=out_specs,
        out_shape=out_shape,
    )


def _make_xform_mid(di, do, t_dtype=jnp.float32):
    out_specs, out_shape = _t_out(do, t_dtype)
    return pl.pallas_call(
        _xform_mid_body,
        grid=(_N // _BM,),
        in_specs=[
            pl.BlockSpec((_NC, _BM, di), lambda i: (0, i, 0)),
            pl.BlockSpec((_BM, di), lambda i: (i, 0)),
            pl.BlockSpec((1, di), lambda i: (0, 0)),
            pl.BlockSpec((_B, di, do), lambda i: (0, 0, 0)),
            pl.BlockSpec(memory_space=pltpu.SMEM),
            pl.BlockSpec((di, do), lambda i: (0, 0)),
        ],
        out_specs=out_specs,
        out_shape=out_shape,
    )


def _make_final(do):
    return pl.pallas_call(
        _final_body,
        grid=(_N // _BM,),
        in_specs=[
            pl.BlockSpec((_NC, _BM, do), lambda i: (0, i, 0)),
            pl.BlockSpec((_BM, do), lambda i: (i, 0)),
            pl.BlockSpec((1, do), lambda i: (0, 0)),
        ],
        out_specs=pl.BlockSpec((_BM, do), lambda i: (i, 0)),
        out_shape=jax.ShapeDtypeStruct((_N, do), jnp.float32),
    )


def _make_sc_agg(do):
    nsl = do // 16
    mesh = plsc.VectorSubcoreMesh(core_axis_name="c", subcore_axis_name="s",
                                  num_cores=_NC, num_subcores=_NS)

    @functools.partial(
        pl.kernel,
        out_type=jax.ShapeDtypeStruct((_NC, _N, do), jnp.float32),
        mesh=mesh,
        scratch_types=[
            pltpu.VMEM((4, 2, _K), jnp.int32),
            pltpu.VMEM((4, _K), jnp.int32),
            pltpu.VMEM((_K, do), jnp.float32),
            pltpu.VMEM((_K, do), jnp.float32),
            pltpu.VMEM((_K, do), jnp.float32),
            pltpu.VMEM((_K, do), jnp.float32),
            pltpu.VMEM_SHARED((_N, do), jnp.float32),
            pltpu.SemaphoreType.DMA,
            pltpu.SemaphoreType.DMA,
            pltpu.SemaphoreType.DMA,
            pltpu.SemaphoreType.DMA,
            pltpu.SemaphoreType.DMA,
            pltpu.SemaphoreType.DMA,
            pltpu.SemaphoreType.DMA,
            pltpu.SemaphoreType.DMA,
            pltpu.SemaphoreType.DMA,
            pltpu.SemaphoreType.DMA,
            pltpu.SemaphoreType.DMA,
            pltpu.SemaphoreType.DMA,
        ],
        compiler_params=pltpu.CompilerParams(use_tc_tiling_on_sc=False,
                                             needs_layout_passes=False),
    )
    def agg(table, metas, zeros, out, meta_v, dst_v,
            rows0, rows1, rows2, rows3, acc_sh,
            gsem0, gsem1, gsem2, gsem3, ssem0, ssem1, ssem2, ssem3,
            msem0, msem1, msem2, msem3):
        c = lax.axis_index("c")
        s = lax.axis_index("s")
        wid = s * _NC + c
        rowbase = wid * _CPT

        rows = (rows0, rows1, rows2, rows3)
        gsem = (gsem0, gsem1, gsem2, gsem3)
        ssem = (ssem0, ssem1, ssem2, ssem3)
        msem = (msem0, msem1, msem2, msem3)

        pltpu.sync_copy(zeros.at[pl.ds(s * _RPT, _RPT)],
                        acc_sh.at[pl.ds(s * _RPT, _RPT)])

        @pl.when(s == _NS - 1)
        def _zero_rem():
            pltpu.sync_copy(zeros.at[pl.ds(_NS * _RPT, _RREM)],
                            acc_sh.at[pl.ds(_NS * _RPT, _RREM)])

        plsc.subcore_barrier()

        def issue_meta(ci, slot):
            pltpu.async_copy(metas.at[rowbase + ci], meta_v.at[slot],
                             msem[slot])

        def wait_meta(slot):
            pltpu.make_async_copy(metas.at[0], meta_v.at[slot],
                                  msem[slot]).wait()

        def issue_gather(slot):
            pltpu.async_copy(table.at[meta_v.at[slot, 0]], rows[slot],
                             gsem[slot])

        def wait_gather(b):
            pltpu.make_async_copy(table.at[meta_v.at[0, 0]], rows[b],
                                  gsem[b]).wait()

        def issue_scatter(b):
            pltpu.async_copy(rows[b], acc_sh.at[dst_v.at[b]], ssem[b],
                             add=True)

        def wait_scatter(b):
            pltpu.make_async_copy(rows[b], acc_sh.at[dst_v.at[0]],
                                  ssem[b]).wait()

        def scale_and_unpack(b):
            buf = rows[b]

            def rowgroup(g, carry):
                pk = meta_v[b, 1, pl.ds(g * 16, 16)]
                dst_v[b, pl.ds(g * 16, 16)] = pk & jnp.int32(0xFFFF)
                nv = plsc.bitcast(pk & jnp.int32(-65536), jnp.float32)
                for t in range(16):
                    i_row = g * 16 + t
                    sn = nv[t]
                    for j in range(nsl):
                        sl = pl.ds(j * 16, 16)
                        buf[i_row, sl] = buf[i_row, sl] * sn
                return carry

            lax.fori_loop(0, _K // 16, rowgroup, 0)

        for slot in range(4):
            issue_meta(slot, slot)
        wait_meta(0)
        issue_gather(0)
        wait_meta(1)
        issue_gather(1)

        def quad(j, carry):
            c0 = j * 4
            for b in range(4):
                ci = c0 + b
                ob = (b + 2) % 4
                if b < 2:
                    @pl.when(j >= 1)
                    def _():
                        wait_scatter(ob)

                    wait_meta(ob)
                    issue_gather(ob)
                elif b == 2:
                    wait_scatter(ob)
                    wait_meta(ob)
                    issue_gather(ob)
                else:
                    wait_scatter(ob)

                    @pl.when(j <= (_CPT - 1) // 4 - 2)
                    def _():
                        wait_meta(ob)
                        issue_gather(ob)
                wait_gather(b)
                scale_and_unpack(b)
                issue_scatter(b)

                @pl.when(ci + 4 <= _CPT - 1)
                def _():
                    issue_meta(ci + 4, b)
            return carry

        lax.fori_loop(0, (_CPT - 1) // 4, quad, 0)

        wait_gather(0)
        scale_and_unpack(0)
        pltpu.sync_copy(rows[0], acc_sh.at[dst_v.at[0]], add=True)
        wait_scatter(2)
        wait_scatter(3)

        plsc.subcore_barrier()
        pltpu.sync_copy(acc_sh.at[pl.ds(s * _RPT, _RPT)],
                        out.at[c, pl.ds(s * _RPT, _RPT)])

        @pl.when(s == _NS - 1)
        def _out_rem():
            pltpu.sync_copy(acc_sh.at[pl.ds(_NS * _RPT, _RREM)],
                            out.at[c, pl.ds(_NS * _RPT, _RREM)])

    return agg


_xform0 = _make_xform_first(128, 128)
_xform1 = _make_xform_mid(128, 128)
_xform2 = _make_xform_mid(128, 16)
_final = _make_final(16)
_make_sc_agg = functools.lru_cache(maxsize=None)(_make_sc_agg)


def kernel(x, edge_index, edge_type, edge_norm,
           W0, C0, LW0, b0, W1, C1, LW1, b1, W2, C2, LW2, b2):
    src = edge_index[0].astype(jnp.int32)
    dst = edge_index[1].astype(jnp.int32)
    et = edge_type.astype(jnp.int32)
    flat_idx = (et * _N + src).reshape(_E // _K, _K)
    nbits = lax.bitcast_convert_type(
        edge_norm.reshape(-1).astype(jnp.bfloat16), jnp.uint16)
    packed = (nbits.astype(jnp.uint32) << 16) | dst.astype(jnp.uint32)
    packed = lax.bitcast_convert_type(packed, jnp.int32).reshape(
        _E // _K, _K)
    metas = jnp.stack([flat_idx, packed], axis=1)
    z128 = jnp.zeros((_N, 128), jnp.float32)
    z16 = jnp.zeros((_N, 16), jnp.float32)

    sc_agg_128 = _make_sc_agg(128)
    sc_agg_16 = _make_sc_agg(16)

    t0, lp0 = _xform0(x, W0, C0, LW0)
    acc0 = sc_agg_128(t0.reshape(_R * _N, 128), metas, z128)

    t1, lp1 = _xform1(acc0, lp0, b0.reshape(1, -1), W1, C1, LW1)
    acc1 = sc_agg_128(t1.reshape(_R * _N, 128), metas, z128)

    t2, lp2 = _xform2(acc1, lp1, b1.reshape(1, -1), W2, C2, LW2)
    acc2 = sc_agg_16(t2.reshape(_R * _N, 16), metas, z16)

    return _final(acc2, lp2, b2.reshape(1, -1))


# speedup vs baseline: 2.5716x; 1.0131x over previous
import functools

import jax
import jax.numpy as jnp
from jax import lax
from jax.experimental import pallas as pl
from jax.experimental.pallas import tpu as pltpu
from jax.experimental.pallas import tpu_sc as plsc

_N = 10000
_E = 320000
_R = 8
_B = 4

_NC = 2
_NS = 16
_NW = _NC * _NS
_EPW = _E // _NW
_K = 80
_CPT = _EPW // _K
_RPT = 624
_RREM = _N - _NS * _RPT

_BM = 2000


def _emit_transform(h, w_ref, c_ref, lw_ref, t_ref, loop_ref):
    bases = [
        jnp.dot(h, w_ref[b], preferred_element_type=jnp.float32)
        for b in range(_B)
    ]
    for r in range(_R):
        acc = c_ref[r, 0] * bases[0]
        for b in range(1, _B):
            acc = acc + c_ref[r, b] * bases[b]
        t_ref[r] = acc.astype(t_ref.dtype)
    loop_ref[...] = jnp.dot(h, lw_ref[...], preferred_element_type=jnp.float32)


def _xform_first_body(x_ref, w_ref, c_ref, lw_ref, t_ref, loop_ref):
    _emit_transform(x_ref[...], w_ref, c_ref, lw_ref, t_ref, loop_ref)


def _xform_mid_body(acc_ref, lp_ref, b_ref, w_ref, c_ref, lw_ref, t_ref,
                    loop_ref):
    h = jnp.maximum(acc_ref[0] + acc_ref[1] + lp_ref[...] + b_ref[...], 0.0)
    _emit_transform(h, w_ref, c_ref, lw_ref, t_ref, loop_ref)


def _final_body(acc_ref, lp_ref, b_ref, o_ref):
    o_ref[...] = acc_ref[0] + acc_ref[1] + lp_ref[...] + b_ref[...]


def _t_out(do, t_dtype):
    return (
        [pl.BlockSpec((_R, _BM, do), lambda i: (0, i, 0)),
         pl.BlockSpec((_BM, do), lambda i: (i, 0))],
        [jax.ShapeDtypeStruct((_R, _N, do), t_dtype),
         jax.ShapeDtypeStruct((_N, do), jnp.float32)],
    )


def _make_xform_first(di, do, t_dtype=jnp.float32):
    out_specs, out_shape = _t_out(do, t_dtype)
    return pl.pallas_call(
        _xform_first_body,
        grid=(_N // _BM,),
        in_specs=[
            pl.BlockSpec((_BM, di), lambda i: (i, 0)),
            pl.BlockSpec((_B, di, do), lambda i: (0, 0, 0)),
            pl.BlockSpec(memory_space=pltpu.SMEM),
            pl.BlockSpec((di, do), lambda i: (0, 0)),
        ],
        out_specs=out_specs,
        out_shape=out_shape,
    )


def _make_xform_mid(di, do, t_dtype=jnp.float32):
    out_specs, out_shape = _t_out(do, t_dtype)
    return pl.pallas_call(
        _xform_mid_body,
        grid=(_N // _BM,),
        in_specs=[
            pl.BlockSpec((_NC, _BM, di), lambda i: (0, i, 0)),
            pl.BlockSpec((_BM, di), lambda i: (i, 0)),
            pl.BlockSpec((1, di), lambda i: (0, 0)),
            pl.BlockSpec((_B, di, do), lambda i: (0, 0, 0)),
            pl.BlockSpec(memory_space=pltpu.SMEM),
            pl.BlockSpec((di, do), lambda i: (0, 0)),
        ],
        out_specs=out_specs,
        out_shape=out_shape,
    )


def _make_final(do):
    return pl.pallas_call(
        _final_body,
        grid=(_N // _BM,),
        in_specs=[
            pl.BlockSpec((_NC, _BM, do), lambda i: (0, i, 0)),
            pl.BlockSpec((_BM, do), lambda i: (i, 0)),
            pl.BlockSpec((1, do), lambda i: (0, 0)),
        ],
        out_specs=pl.BlockSpec((_BM, do), lambda i: (i, 0)),
        out_shape=jax.ShapeDtypeStruct((_N, do), jnp.float32),
    )


def _make_sc_agg(do):
    nsl = do // 16
    mesh = plsc.VectorSubcoreMesh(core_axis_name="c", subcore_axis_name="s",
                                  num_cores=_NC, num_subcores=_NS)

    @functools.partial(
        pl.kernel,
        out_type=jax.ShapeDtypeStruct((_NC, _N, do), jnp.float32),
        mesh=mesh,
        scratch_types=[
            pltpu.VMEM((4, 2, _K), jnp.int32),
            pltpu.VMEM((4, _K), jnp.int32),
            pltpu.VMEM((_K, do), jnp.float32),
            pltpu.VMEM((_K, do), jnp.float32),
            pltpu.VMEM((_K, do), jnp.float32),
            pltpu.VMEM((_K, do), jnp.float32),
            pltpu.VMEM_SHARED((_N, do), jnp.float32),
            pltpu.SemaphoreType.DMA,
            pltpu.SemaphoreType.DMA,
            pltpu.SemaphoreType.DMA,
            pltpu.SemaphoreType.DMA,
            pltpu.SemaphoreType.DMA,
            pltpu.SemaphoreType.DMA,
            pltpu.SemaphoreType.DMA,
            pltpu.SemaphoreType.DMA,
            pltpu.SemaphoreType.DMA,
            pltpu.SemaphoreType.DMA,
            pltpu.SemaphoreType.DMA,
            pltpu.SemaphoreType.DMA,
        ],
        compiler_params=pltpu.CompilerParams(use_tc_tiling_on_sc=False,
                                             needs_layout_passes=False),
    )
    def agg(table, metas, zeros, out, meta_v, dst_v,
            rows0, rows1, rows2, rows3, acc_sh,
            gsem0, gsem1, gsem2, gsem3, ssem0, ssem1, ssem2, ssem3,
            msem0, msem1, msem2, msem3):
        c = lax.axis_index("c")
        s = lax.axis_index("s")
        wid = s * _NC + c
        rowbase = wid * _CPT

        rows = (rows0, rows1, rows2, rows3)
        gsem = (gsem0, gsem1, gsem2, gsem3)
        ssem = (ssem0, ssem1, ssem2, ssem3)
        msem = (msem0, msem1, msem2, msem3)

        pltpu.sync_copy(zeros.at[pl.ds(s * _RPT, _RPT)],
                        acc_sh.at[pl.ds(s * _RPT, _RPT)])

        @pl.when(s == _NS - 1)
        def _zero_rem():
            pltpu.sync_copy(zeros.at[pl.ds(_NS * _RPT, _RREM)],
                            acc_sh.at[pl.ds(_NS * _RPT, _RREM)])

        plsc.subcore_barrier()

        def issue_meta(ci, slot):
            pltpu.async_copy(metas.at[rowbase + ci], meta_v.at[slot],
                             msem[slot])

        def wait_meta(slot):
            pltpu.make_async_copy(metas.at[0], meta_v.at[slot],
                                  msem[slot]).wait()

        def issue_gather(slot):
            pltpu.async_copy(table.at[meta_v.at[slot, 0]], rows[slot],
                             gsem[slot])

        def wait_gather(b):
            pltpu.make_async_copy(table.at[meta_v.at[0, 0]], rows[b],
                                  gsem[b]).wait()

        def issue_scatter(b):
            pltpu.async_copy(rows[b], acc_sh.at[dst_v.at[b]], ssem[b],
                             add=True)

        def wait_scatter(b):
            pltpu.make_async_copy(rows[b], acc_sh.at[dst_v.at[0]],
                                  ssem[b]).wait()

        def scale_and_unpack(b):
            buf = rows[b]

            def rowgroup(g, carry):
                pk = meta_v[b, 1, pl.ds(g * 16, 16)]
                dst_v[b, pl.ds(g * 16, 16)] = pk & jnp.int32(0xFFFF)
                nv = plsc.bitcast(pk & jnp.int32(-65536), jnp.float32)
                for t in range(16):
                    i_row = g * 16 + t
                    sn = nv[t]
                    for j in range(nsl):
                        sl = pl.ds(j * 16, 16)
                        buf[i_row, sl] = buf[i_row, sl] * sn
                return carry

            lax.fori_loop(0, _K // 16, rowgroup, 0)

        for slot in range(4):
            issue_meta(slot, slot)
        wait_meta(0)
        issue_gather(0)
        wait_meta(1)
        issue_gather(1)

        def quad(j, carry):
            c0 = j * 4
            for b in range(4):
                ci = c0 + b
                ob = (b + 2) % 4
                if b < 2:
                    @pl.when(j >= 1)
                    def _():
                        wait_scatter(ob)

                    wait_meta(ob)
                    issue_gather(ob)
                elif b == 2:
                    wait_scatter(ob)
                    wait_meta(ob)
                    issue_gather(ob)
                else:
                    wait_scatter(ob)

                    @pl.when(j <= (_CPT - 1) // 4 - 2)
                    def _():
                        wait_meta(ob)
                        issue_gather(ob)
                wait_gather(b)
                scale_and_unpack(b)
                issue_scatter(b)

                @pl.when(ci + 4 <= _CPT - 1)
                def _():
                    issue_meta(ci + 4, b)
            return carry

        lax.fori_loop(0, (_CPT - 1) // 4, quad, 0)

        wait_gather(0)
        scale_and_unpack(0)
        pltpu.sync_copy(rows[0], acc_sh.at[dst_v.at[0]], add=True)
        wait_scatter(2)
        wait_scatter(3)

        plsc.subcore_barrier()
        pltpu.sync_copy(acc_sh.at[pl.ds(s * _RPT, _RPT)],
                        out.at[c, pl.ds(s * _RPT, _RPT)])

        @pl.when(s == _NS - 1)
        def _out_rem():
            pltpu.sync_copy(acc_sh.at[pl.ds(_NS * _RPT, _RREM)],
                            out.at[c, pl.ds(_NS * _RPT, _RREM)])

    return agg


_xform0 = _make_xform_first(128, 128)
_xform1 = _make_xform_mid(128, 128)
_xform2 = _make_xform_mid(128, 16)
_final = _make_final(16)
_make_sc_agg = functools.lru_cache(maxsize=None)(_make_sc_agg)


def kernel(x, edge_index, edge_type, edge_norm,
           W0, C0, LW0, b0, W1, C1, LW1, b1, W2, C2, LW2, b2):
    src = edge_index[0].astype(jnp.int32)
    dst = edge_index[1].astype(jnp.int32)
    et = edge_type.astype(jnp.int32)
    flat_idx = (et * _N + src).reshape(_E // _K, _K)
    nbits = lax.bitcast_convert_type(
        edge_norm.reshape(-1).astype(jnp.bfloat16), jnp.uint16)
    packed = (nbits.astype(jnp.uint32) << 16) | dst.astype(jnp.uint32)
    packed = lax.bitcast_convert_type(packed, jnp.int32).reshape(
        _E // _K, _K)
    metas = jnp.stack([flat_idx, packed], axis=1)
    z128 = jnp.zeros((_N, 128), jnp.float32)
    z16 = jnp.zeros((_N, 16), jnp.float32)

    sc_agg_128 = _make_sc_agg(128)
    sc_agg_16 = _make_sc_agg(16)

    t0, lp0 = _xform0(x, W0, C0, LW0)
    acc0 = sc_agg_128(t0.reshape(_R * _N, 128), metas, z128)

    t1, lp1 = _xform1(acc0, lp0, b0.reshape(1, -1), W1, C1, LW1)
    acc1 = sc_agg_128(t1.reshape(_R * _N, 128), metas, z128)

    t2, lp2 = _xform2(acc1, lp1, b1.reshape(1, -1), W2, C2, LW2)
    acc2 = sc_agg_16(t2.reshape(_R * _N, 16), metas, z16)

    return _final(acc2, lp2, b2.reshape(1, -1))
